# baseline (speedup 1.0000x reference)
import functools

import jax
import jax.numpy as jnp
from jax import lax
from jax.experimental import pallas as pl
from jax.experimental.pallas import tpu as pltpu
from jax.experimental.pallas import tpu_sc as plsc

N = 10000
E = 320000
D = 128
BGRAPH = 64
NC, NS = 2, 16
NW = NC * NS
NP = 10240
EP = 327680
EC = 80
ROWS_PER_SUB = NP // NS
NBUF = 4

_MESH = plsc.VectorSubcoreMesh(
    core_axis_name="c", subcore_axis_name="s", num_cores=NC, num_subcores=NS)


@functools.partial(
    pl.kernel,
    out_type=[
        jax.ShapeDtypeStruct((NP, D), jnp.float32),
        jax.ShapeDtypeStruct((NC, NP, 64), jnp.float32),
    ],
    mesh=_MESH,
    compiler_params=pltpu.CompilerParams(use_tc_tiling_on_sc=False),
    scratch_types=[
        pltpu.VMEM((5, 64), jnp.int32),
        pltpu.VMEM((64, D), jnp.float32),
        pltpu.VMEM((EC, 128), jnp.int32),
        pltpu.VMEM((128, 64), jnp.float32),
        pltpu.VMEM_SHARED((NP, 64), jnp.float32),
        pltpu.SemaphoreType.DMA,
    ],
)
def _sc_embed_deg(table, embidx, dstidx, zeros_deg, ones_col,
                  x_out, deg_out, idx_v, rows_v, dstbuf, ones_v, deg_acc, sem):
    c = lax.axis_index("c")
    s = lax.axis_index("s")
    w = s * NC + c
    pltpu.sync_copy(zeros_deg, deg_acc.at[pl.ds(s * ROWS_PER_SUB, ROWS_PER_SUB)])
    pltpu.sync_copy(embidx.at[w], idx_v)
    for j in range(5):
        pltpu.async_copy(table.at[idx_v.at[j]], rows_v, sem).wait()
        pltpu.sync_copy(rows_v, x_out.at[pl.ds(w * 320 + j * 64, 64)])
    pltpu.sync_copy(dstidx.at[w], dstbuf)
    pltpu.sync_copy(ones_col, ones_v)
    plsc.subcore_barrier()

    @pl.loop(0, EC)
    def _deg_loop(j):
        pltpu.sync_copy(ones_v, deg_acc.at[dstbuf.at[j]], add=True)

    plsc.subcore_barrier()
    sl = pl.ds(s * ROWS_PER_SUB, ROWS_PER_SUB)
    pltpu.sync_copy(deg_acc.at[sl], deg_out.at[c, sl])


DH = D // 2


TOTC = 2 * EC
CH0 = 144
CH1 = TOTC - CH0


@functools.partial(
    pl.kernel,
    out_type=jax.ShapeDtypeStruct((NC, 2, NP, DH), jnp.float32),
    mesh=_MESH,
    compiler_params=pltpu.CompilerParams(use_tc_tiling_on_sc=False),
    scratch_types=[
        pltpu.VMEM((CH0, 128), jnp.int32),
        pltpu.VMEM((CH0, 128), jnp.int32),
        pltpu.VMEM((NBUF, 128, DH), jnp.float32),
        pltpu.VMEM_SHARED((NP, DH), jnp.float32),
        pltpu.SemaphoreType.DMA,
        pltpu.SemaphoreType.DMA,
    ],
)
def _sc_edge_pass(h_lo, h_hi, srcidx, dstidx, zeros_rows,
                  out, srcbuf, dstbuf, rows, acc, sem_g, sem_s):
    c = lax.axis_index("c")
    s = lax.axis_index("s")
    mych = jnp.where(c == 0, CH0, CH1)
    ngroups = jnp.where(c == 0, CH0 // NBUF, CH1 // NBUF)
    sl = pl.ds(s * ROWS_PER_SUB, ROWS_PER_SUB)
    pltpu.sync_copy(zeros_rows, acc.at[sl])

    @pl.when(c == 0)
    def _stage0():
        pltpu.sync_copy(srcidx.at[s, pl.ds(0, CH0)], srcbuf)
        pltpu.sync_copy(dstidx.at[s, pl.ds(0, CH0)], dstbuf)

    @pl.when(c == 1)
    def _stage1():
        pltpu.sync_copy(srcidx.at[s, pl.ds(CH0, CH1)], srcbuf.at[pl.ds(0, CH1)])
        pltpu.sync_copy(dstidx.at[s, pl.ds(CH0, CH1)], dstbuf.at[pl.ds(0, CH1)])

    plsc.subcore_barrier()
    for half, h in ((0, h_lo), (1, h_hi)):
        for b in range(NBUF - 1):
            pltpu.async_copy(h.at[srcbuf.at[b]], rows.at[b], sem_g)

        @pl.loop(0, ngroups)
        def _edge_loop(g):
            for b in range(NBUF):
                j = g * NBUF + b
                pltpu.make_async_copy(h.at[srcbuf.at[j]], rows.at[b], sem_g).wait()
                pltpu.async_copy(rows.at[b], acc.at[dstbuf.at[j]], sem_s, add=True)
                if b == 0:
                    @pl.when(g > 0)
                    def _wait_prev():
                        pltpu.make_async_copy(
                            rows.at[NBUF - 1],
                            acc.at[dstbuf.at[g * NBUF - 1]], sem_s).wait()
                else:
                    pltpu.make_async_copy(
                        rows.at[b - 1], acc.at[dstbuf.at[j - 1]], sem_s).wait()

                @pl.when(j + NBUF - 1 < mych)
                def _prefetch():
                    pltpu.async_copy(
                        h.at[srcbuf.at[j + NBUF - 1]],
                        rows.at[(j + NBUF - 1) % NBUF], sem_g)

        pltpu.make_async_copy(
            rows.at[NBUF - 1], acc.at[dstbuf.at[mych - 1]], sem_s).wait()
        plsc.subcore_barrier()
        pltpu.sync_copy(acc.at[sl], out.at[c, half, sl])
        if half == 0:
            pltpu.sync_copy(zeros_rows, acc.at[sl])
            plsc.subcore_barrier()


def _halves(p_ref, lo_ref, hi_ref):
    lo = p_ref[0, 0] + p_ref[1, 0] + lo_ref[...]
    hi = p_ref[0, 1] + p_ref[1, 1] + hi_ref[...]
    return jnp.concatenate([lo, hi], axis=1)


def _tc_prep1(x_ref, degp_ref, w_ref, b_ref, lo_ref, hi_ref, dinv_ref):
    deg = degp_ref[0, :, 0:1] + degp_ref[1, :, 0:1] + 1.0
    dinv = lax.rsqrt(jnp.maximum(deg, 1.0))
    dinv_ref[...] = dinv
    h = jnp.dot(x_ref[...], w_ref[...], preferred_element_type=jnp.float32)
    h = (h + b_ref[...]) * dinv
    lo_ref[...] = h[:, :DH]
    hi_ref[...] = h[:, DH:]


def _tc_prep2(p_ref, h1lo_ref, h1hi_ref, dinv_ref, w_ref, b_ref,
              lo_ref, hi_ref):
    dinv = dinv_ref[...]
    x1 = jnp.maximum(dinv * _halves(p_ref, h1lo_ref, h1hi_ref), 0.0)
    h = jnp.dot(x1, w_ref[...], preferred_element_type=jnp.float32)
    h = (h + b_ref[...]) * dinv
    lo_ref[...] = h[:, :DH]
    hi_ref[...] = h[:, DH:]


def _tc_wx(q_ref, h2lo_ref, h2hi_ref, dinv_ref, sw_ref, wx_ref):
    x2 = dinv_ref[...] * _halves(q_ref, h2lo_ref, h2hi_ref)
    wx_ref[...] = x2 * sw_ref[...]


def _tc_final(wx_ref, batch_ref,
              wi_ref, wh_ref, bi_ref, bh_ref, lw_ref, lb_ref, out_ref):
    f32 = jnp.float32
    wx = wx_ref[...]
    bids = lax.broadcasted_iota(jnp.int32, (NP, BGRAPH), 1)
    mt = (batch_ref[...] == bids).astype(f32)

    def dot0(a, b):
        return lax.dot_general(a, b, (((0,), (0,)), ((), ())),
                               preferred_element_type=f32)

    sto = dot0(mt, wx)
    q_star = jnp.zeros((BGRAPH, 2 * D), f32)
    hh = jnp.zeros((BGRAPH, D), f32)
    cc = jnp.zeros((BGRAPH, D), f32)
    for _ in range(2):
        gates = (jnp.dot(q_star, wi_ref[...], preferred_element_type=f32)
                 + jnp.dot(hh, wh_ref[...], preferred_element_type=f32)
                 + bi_ref[...] + bh_ref[...])
        ii = jax.nn.sigmoid(gates[:, 0:D])
        ff = jax.nn.sigmoid(gates[:, D:2 * D])
        gg = jnp.tanh(gates[:, 2 * D:3 * D])
        oo = jax.nn.sigmoid(gates[:, 3 * D:4 * D])
        cc = ff * cc + ii * gg
        hh = oo * jnp.tanh(cc)
        patt = lax.dot_general(wx, hh, (((1,), (1,)), ((), ())),
                               preferred_element_type=f32)
        e = jnp.sum(mt * patt, axis=1, keepdims=True)
        emax = jnp.max(jnp.where(mt > 0, patt, -jnp.inf), axis=0, keepdims=True)
        emax = jnp.where(jnp.isneginf(emax), 0.0, emax)
        eg = jnp.sum(mt * emax, axis=1, keepdims=True)
        ee = jnp.exp(e - eg)
        denom = lax.dot_general(ee, mt, (((0,), (0,)), ((), ())),
                                preferred_element_type=f32)
        dg = jnp.sum(mt * denom, axis=1, keepdims=True)
        a = ee / (dg + 1e-16)
        r = dot0(mt, wx * a)
        q_star = jnp.concatenate([hh, r], axis=1)
    out = jnp.dot(q_star, lw_ref[...], preferred_element_type=f32)
    out_ref[...] = out + lb_ref[...] + sto


def kernel(sto_x, sto_edge_index, sto_weight, sto_batch, weight,
           W1, b1, W2, b2, lstm_Wi, lstm_Wh, lstm_bi, lstm_bh, lin_W, lin_b):
    f32 = jnp.float32
    i32 = jnp.int32
    embidx = jnp.pad(sto_x.astype(i32) - 1, (0, NP - N)).reshape(NW, 5, 64)
    src = jnp.pad(sto_edge_index[0].astype(i32), (0, EP - E))
    dst = jnp.pad(sto_edge_index[1].astype(i32), (0, EP - E),
                  constant_values=N)
    src_e = src.reshape(NS, TOTC, 128)
    dst_e = dst.reshape(NS, TOTC, 128)
    dst = dst.reshape(NW, EC, 128)
    sw = jnp.pad(sto_weight.astype(f32), (0, NP - N)).reshape(NP, 1)
    batch = jnp.pad(sto_batch.astype(i32), (0, NP - N),
                    constant_values=127).reshape(NP, 1)
    zeros_deg = jnp.zeros((ROWS_PER_SUB, 64), f32)
    zeros_rows = jnp.zeros((ROWS_PER_SUB, DH), f32)
    ones_col = jnp.ones((128, 64), f32)

    x, degp = _sc_embed_deg(weight.astype(f32), embidx, dst, zeros_deg, ones_col)

    h1lo, h1hi, dinv = pl.pallas_call(
        _tc_prep1,
        out_shape=[jax.ShapeDtypeStruct((NP, DH), f32),
                   jax.ShapeDtypeStruct((NP, DH), f32),
                   jax.ShapeDtypeStruct((NP, 1), f32)],
    )(x, degp, W1, b1.reshape(1, D))

    p1 = _sc_edge_pass(h1lo, h1hi, src_e, dst_e, zeros_rows)

    h2lo, h2hi = pl.pallas_call(
        _tc_prep2,
        out_shape=[jax.ShapeDtypeStruct((NP, DH), f32),
                   jax.ShapeDtypeStruct((NP, DH), f32)],
    )(p1, h1lo, h1hi, dinv, W2, b2.reshape(1, D))

    p2 = _sc_edge_pass(h2lo, h2hi, src_e, dst_e, zeros_rows)

    wx = pl.pallas_call(
        _tc_wx,
        out_shape=jax.ShapeDtypeStruct((NP, D), f32),
    )(p2, h2lo, h2hi, dinv, sw)

    out = pl.pallas_call(
        _tc_final,
        out_shape=jax.ShapeDtypeStruct((BGRAPH, D), f32),
    )(wx, batch, lstm_Wi, lstm_Wh,
      lstm_bi.reshape(1, 4 * D), lstm_bh.reshape(1, 4 * D),
      lin_W, lin_b.reshape(1, D))
    return out


# speedup vs baseline: 1.0840x; 1.0840x over previous
import functools

import jax
import jax.numpy as jnp
from jax import lax
from jax.experimental import pallas as pl
from jax.experimental.pallas import tpu as pltpu
from jax.experimental.pallas import tpu_sc as plsc

N = 10000
E = 320000
D = 128
BGRAPH = 64
NC, NS = 2, 16
NW = NC * NS
NP = 10240
EP = 327680
EC = 80
ROWS_PER_SUB = NP // NS
NBUF = 4

_MESH = plsc.VectorSubcoreMesh(
    core_axis_name="c", subcore_axis_name="s", num_cores=NC, num_subcores=NS)


@functools.partial(
    pl.kernel,
    out_type=[
        jax.ShapeDtypeStruct((NP, D), jnp.float32),
        jax.ShapeDtypeStruct((NC, NP, 64), jnp.float32),
    ],
    mesh=_MESH,
    compiler_params=pltpu.CompilerParams(use_tc_tiling_on_sc=False),
    scratch_types=[
        pltpu.VMEM((5, 64), jnp.int32),
        pltpu.VMEM((64, D), jnp.float32),
        pltpu.VMEM((EC, 128), jnp.int32),
        pltpu.VMEM((128, 64), jnp.float32),
        pltpu.VMEM((128, 64), jnp.float32),
        pltpu.VMEM_SHARED((NP, 64), jnp.float32),
        pltpu.SemaphoreType.DMA,
    ],
)
def _sc_embed_deg(table, embidx, dstidx, zeros_deg, ones_col,
                  x_out, deg_out, idx_v, rows_v, dstbuf, ones_v, zbuf, deg_acc, sem):
    c = lax.axis_index("c")
    s = lax.axis_index("s")
    w = s * NC + c
    pltpu.sync_copy(zeros_deg, zbuf)
    for k in range(ROWS_PER_SUB // 128):
        pltpu.sync_copy(zbuf, deg_acc.at[pl.ds(s * ROWS_PER_SUB + k * 128, 128)])
    pltpu.sync_copy(embidx.at[w], idx_v)
    for j in range(5):
        pltpu.async_copy(table.at[idx_v.at[j]], rows_v, sem).wait()
        pltpu.sync_copy(rows_v, x_out.at[pl.ds(w * 320 + j * 64, 64)])
    pltpu.sync_copy(dstidx.at[w], dstbuf)
    pltpu.sync_copy(ones_col, ones_v)
    plsc.subcore_barrier()

    @pl.loop(0, EC)
    def _deg_loop(j):
        pltpu.sync_copy(ones_v, deg_acc.at[dstbuf.at[j]], add=True)

    plsc.subcore_barrier()
    sl = pl.ds(s * ROWS_PER_SUB, ROWS_PER_SUB)
    pltpu.sync_copy(deg_acc.at[sl], deg_out.at[c, sl])


DH = D // 2


TOTC = 2 * EC
CH0 = 128
CH1 = TOTC - CH0


@functools.partial(
    pl.kernel,
    out_type=jax.ShapeDtypeStruct((NC, 2, NP, DH), jnp.float32),
    mesh=_MESH,
    compiler_params=pltpu.CompilerParams(use_tc_tiling_on_sc=False),
    scratch_types=[
        pltpu.VMEM((CH0, 128), jnp.int32),
        pltpu.VMEM((CH0, 128), jnp.int32),
        pltpu.VMEM((NBUF, 128, DH), jnp.float32),
        pltpu.VMEM((128, DH), jnp.float32),
        pltpu.VMEM_SHARED((NP, DH), jnp.float32),
        pltpu.SemaphoreType.DMA,
        pltpu.SemaphoreType.DMA,
    ],
)
def _sc_edge_pass(h_lo, h_hi, srcidx, dstidx, zeros_rows,
                  out, srcbuf, dstbuf, rows, zbuf, acc, sem_g, sem_s):
    c = lax.axis_index("c")
    s = lax.axis_index("s")
    mych = jnp.where(c == 0, CH0, CH1)
    ngroups = jnp.where(c == 0, CH0 // NBUF, CH1 // NBUF)
    sl = pl.ds(s * ROWS_PER_SUB, ROWS_PER_SUB)
    pltpu.sync_copy(zeros_rows, zbuf)
    for k in range(ROWS_PER_SUB // 128):
        pltpu.sync_copy(zbuf, acc.at[pl.ds(s * ROWS_PER_SUB + k * 128, 128)])

    @pl.when(c == 0)
    def _stage0():
        pltpu.sync_copy(srcidx.at[s, pl.ds(0, CH0)], srcbuf)
        pltpu.sync_copy(dstidx.at[s, pl.ds(0, CH0)], dstbuf)

    @pl.when(c == 1)
    def _stage1():
        pltpu.sync_copy(srcidx.at[s, pl.ds(CH0, CH1)], srcbuf.at[pl.ds(0, CH1)])
        pltpu.sync_copy(dstidx.at[s, pl.ds(CH0, CH1)], dstbuf.at[pl.ds(0, CH1)])

    plsc.subcore_barrier()
    for half, h in ((0, h_lo), (1, h_hi)):
        for b in range(NBUF - 1):
            pltpu.async_copy(h.at[srcbuf.at[b]], rows.at[b], sem_g)

        @pl.loop(0, ngroups)
        def _edge_loop(g):
            for b in range(NBUF):
                j = g * NBUF + b
                pltpu.make_async_copy(h.at[srcbuf.at[j]], rows.at[b], sem_g).wait()
                pltpu.async_copy(rows.at[b], acc.at[dstbuf.at[j]], sem_s, add=True)
                if b == 0:
                    @pl.when(g > 0)
                    def _wait_prev():
                        pltpu.make_async_copy(
                            rows.at[NBUF - 1],
                            acc.at[dstbuf.at[g * NBUF - 1]], sem_s).wait()
                else:
                    pltpu.make_async_copy(
                        rows.at[b - 1], acc.at[dstbuf.at[j - 1]], sem_s).wait()

                @pl.when(j + NBUF - 1 < mych)
                def _prefetch():
                    pltpu.async_copy(
                        h.at[srcbuf.at[j + NBUF - 1]],
                        rows.at[(j + NBUF - 1) % NBUF], sem_g)

        pltpu.make_async_copy(
            rows.at[NBUF - 1], acc.at[dstbuf.at[mych - 1]], sem_s).wait()
        plsc.subcore_barrier()
        pltpu.sync_copy(acc.at[sl], out.at[c, half, sl])
        if half == 0:
            for k in range(ROWS_PER_SUB // 128):
                pltpu.sync_copy(zbuf, acc.at[pl.ds(s * ROWS_PER_SUB + k * 128, 128)])
            plsc.subcore_barrier()


def _halves(p_ref, lo_ref, hi_ref):
    lo = p_ref[0, 0] + p_ref[1, 0] + lo_ref[...]
    hi = p_ref[0, 1] + p_ref[1, 1] + hi_ref[...]
    return jnp.concatenate([lo, hi], axis=1)


def _tc_prep1(x_ref, degp_ref, w_ref, b_ref, lo_ref, hi_ref, dinv_ref):
    deg = degp_ref[0, :, 0:1] + degp_ref[1, :, 0:1] + 1.0
    dinv = lax.rsqrt(jnp.maximum(deg, 1.0))
    dinv_ref[...] = dinv
    h = jnp.dot(x_ref[...], w_ref[...], preferred_element_type=jnp.float32)
    h = (h + b_ref[...]) * dinv
    lo_ref[...] = h[:, :DH]
    hi_ref[...] = h[:, DH:]


def _tc_prep2(p_ref, h1lo_ref, h1hi_ref, dinv_ref, w_ref, b_ref,
              lo_ref, hi_ref):
    dinv = dinv_ref[...]
    x1 = jnp.maximum(dinv * _halves(p_ref, h1lo_ref, h1hi_ref), 0.0)
    h = jnp.dot(x1, w_ref[...], preferred_element_type=jnp.float32)
    h = (h + b_ref[...]) * dinv
    lo_ref[...] = h[:, :DH]
    hi_ref[...] = h[:, DH:]


def _tc_wx(q_ref, h2lo_ref, h2hi_ref, dinv_ref, sw_ref, wx_ref):
    x2 = dinv_ref[...] * _halves(q_ref, h2lo_ref, h2hi_ref)
    wx_ref[...] = x2 * sw_ref[...]


def _tc_final(wx_ref, batch_ref,
              wi_ref, wh_ref, bi_ref, bh_ref, lw_ref, lb_ref, out_ref):
    f32 = jnp.float32
    wx = wx_ref[...]
    bids = lax.broadcasted_iota(jnp.int32, (NP, BGRAPH), 1)
    mt = (batch_ref[...] == bids).astype(f32)

    def dot0(a, b):
        return lax.dot_general(a, b, (((0,), (0,)), ((), ())),
                               preferred_element_type=f32)

    sto = dot0(mt, wx)
    q_star = jnp.zeros((BGRAPH, 2 * D), f32)
    hh = jnp.zeros((BGRAPH, D), f32)
    cc = jnp.zeros((BGRAPH, D), f32)
    for _ in range(2):
        gates = (jnp.dot(q_star, wi_ref[...], preferred_element_type=f32)
                 + jnp.dot(hh, wh_ref[...], preferred_element_type=f32)
                 + bi_ref[...] + bh_ref[...])
        ii = jax.nn.sigmoid(gates[:, 0:D])
        ff = jax.nn.sigmoid(gates[:, D:2 * D])
        gg = jnp.tanh(gates[:, 2 * D:3 * D])
        oo = jax.nn.sigmoid(gates[:, 3 * D:4 * D])
        cc = ff * cc + ii * gg
        hh = oo * jnp.tanh(cc)
        patt = lax.dot_general(wx, hh, (((1,), (1,)), ((), ())),
                               preferred_element_type=f32)
        e = jnp.sum(mt * patt, axis=1, keepdims=True)
        emax = jnp.max(jnp.where(mt > 0, patt, -jnp.inf), axis=0, keepdims=True)
        emax = jnp.where(jnp.isneginf(emax), 0.0, emax)
        eg = jnp.sum(mt * emax, axis=1, keepdims=True)
        ee = jnp.exp(e - eg)
        denom = lax.dot_general(ee, mt, (((0,), (0,)), ((), ())),
                                preferred_element_type=f32)
        dg = jnp.sum(mt * denom, axis=1, keepdims=True)
        a = ee / (dg + 1e-16)
        r = dot0(mt, wx * a)
        q_star = jnp.concatenate([hh, r], axis=1)
    out = jnp.dot(q_star, lw_ref[...], preferred_element_type=f32)
    out_ref[...] = out + lb_ref[...] + sto


def kernel(sto_x, sto_edge_index, sto_weight, sto_batch, weight,
           W1, b1, W2, b2, lstm_Wi, lstm_Wh, lstm_bi, lstm_bh, lin_W, lin_b):
    f32 = jnp.float32
    i32 = jnp.int32
    embidx = jnp.pad(sto_x.astype(i32) - 1, (0, NP - N)).reshape(NW, 5, 64)
    src = jnp.pad(sto_edge_index[0].astype(i32), (0, EP - E))
    dst = jnp.pad(sto_edge_index[1].astype(i32), (0, EP - E),
                  constant_values=N)
    src_e = src.reshape(NS, TOTC, 128)
    dst_e = dst.reshape(NS, TOTC, 128)
    dst = dst.reshape(NW, EC, 128)
    sw = jnp.pad(sto_weight.astype(f32), (0, NP - N)).reshape(NP, 1)
    batch = jnp.pad(sto_batch.astype(i32), (0, NP - N),
                    constant_values=127).reshape(NP, 1)
    zeros_deg = jnp.zeros((128, 64), f32)
    zeros_rows = jnp.zeros((128, DH), f32)
    ones_col = jnp.ones((128, 64), f32)

    x, degp = _sc_embed_deg(weight.astype(f32), embidx, dst, zeros_deg, ones_col)

    h1lo, h1hi, dinv = pl.pallas_call(
        _tc_prep1,
        out_shape=[jax.ShapeDtypeStruct((NP, DH), f32),
                   jax.ShapeDtypeStruct((NP, DH), f32),
                   jax.ShapeDtypeStruct((NP, 1), f32)],
    )(x, degp, W1, b1.reshape(1, D))

    p1 = _sc_edge_pass(h1lo, h1hi, src_e, dst_e, zeros_rows)

    h2lo, h2hi = pl.pallas_call(
        _tc_prep2,
        out_shape=[jax.ShapeDtypeStruct((NP, DH), f32),
                   jax.ShapeDtypeStruct((NP, DH), f32)],
    )(p1, h1lo, h1hi, dinv, W2, b2.reshape(1, D))

    p2 = _sc_edge_pass(h2lo, h2hi, src_e, dst_e, zeros_rows)

    wx = pl.pallas_call(
        _tc_wx,
        out_shape=jax.ShapeDtypeStruct((NP, D), f32),
    )(p2, h2lo, h2hi, dinv, sw)

    out = pl.pallas_call(
        _tc_final,
        out_shape=jax.ShapeDtypeStruct((BGRAPH, D), f32),
    )(wx, batch, lstm_Wi, lstm_Wh,
      lstm_bi.reshape(1, 4 * D), lstm_bh.reshape(1, 4 * D),
      lin_W, lin_b.reshape(1, D))
    return out
